# baseline (speedup 1.0000x reference)
import functools

import jax
import jax.numpy as jnp
from jax import lax
from jax.experimental import pallas as pl
from jax.experimental.pallas import tpu as pltpu
from jax.experimental.pallas import tpu_sc as plsc

N = 10000
E = 320000
D = 128

NC = 2
NS = 16
NW = NC * NS
EPW = E // NW
C = 128
FULL = EPW // C
TAIL = EPW - FULL * C
NPAD = 10240
RPT = NPAD // NS
CPT = NPAD // NS

_mesh = plsc.VectorSubcoreMesh(core_axis_name="c", subcore_axis_name="s")


def _make_sc_agg(with_cnt: bool):
  out_type = [jax.ShapeDtypeStruct((NC, NPAD, D), jnp.float32)]
  if with_cnt:
    out_type.append(jax.ShapeDtypeStruct((NC, NPAD), jnp.float32))
  scratch = [
      pltpu.VMEM((C,), jnp.int32),
      pltpu.VMEM((C,), jnp.int32),
      pltpu.VMEM((C, D), jnp.float32),
      pltpu.VMEM((C,), jnp.int32),
      pltpu.VMEM((C,), jnp.int32),
      pltpu.VMEM((C, D), jnp.float32),
      pltpu.VMEM((TAIL,), jnp.int32),
      pltpu.VMEM((TAIL,), jnp.int32),
      pltpu.VMEM((TAIL, D), jnp.float32),
      pltpu.VMEM_SHARED((NPAD, D), jnp.float32),
      pltpu.SemaphoreType.DMA,
      pltpu.SemaphoreType.DMA,
      pltpu.SemaphoreType.DMA,
      pltpu.SemaphoreType.DMA,
  ]
  if with_cnt:
    scratch += [
        pltpu.VMEM((C,), jnp.float32),
        pltpu.VMEM((TAIL,), jnp.float32),
        pltpu.VMEM_SHARED((NPAD,), jnp.float32),
    ]

  def body(src_hbm, dst_hbm, x_hbm, zrows_hbm, *rest):
    if with_cnt:
      (zcnt_hbm, p_hbm, cnt_hbm, sidx0, didx0, rows0, sidx1, didx1, rows1,
       sidx_t, didx_t, rows_t, acc, isem0, isem1, gsem0, gsem1,
       ones, ones_t, cacc) = rest
    else:
      (p_hbm, sidx0, didx0, rows0, sidx1, didx1, rows1,
       sidx_t, didx_t, rows_t, acc, isem0, isem1, gsem0, gsem1) = rest
    c = lax.axis_index("c")
    s = lax.axis_index("s")
    wid = c * NS + s
    base = wid * EPW
    bufs = ((sidx0, didx0, rows0, isem0, gsem0),
            (sidx1, didx1, rows1, isem1, gsem1))

    def idx_start(off, b):
      pltpu.async_copy(src_hbm.at[pl.ds(off, C)], b[0], b[3])
      pltpu.async_copy(dst_hbm.at[pl.ds(off, C)], b[1], b[3])

    def idx_wait(off, b):
      pltpu.make_async_copy(src_hbm.at[pl.ds(off, C)], b[0], b[3]).wait()
      pltpu.make_async_copy(dst_hbm.at[pl.ds(off, C)], b[1], b[3]).wait()

    def gather_start(b):
      pltpu.async_copy(x_hbm.at[b[0]], b[2], b[4])

    def gather_wait(b):
      pltpu.make_async_copy(x_hbm.at[b[0]], b[2], b[4]).wait()

    def scatter(b):
      pltpu.sync_copy(b[2], acc.at[b[1]], add=True)
      if with_cnt:
        pltpu.sync_copy(ones, cacc.at[b[1]], add=True)

    row0 = s * RPT
    pltpu.sync_copy(zrows_hbm, acc.at[pl.ds(row0, RPT)])
    if with_cnt:
      pltpu.sync_copy(zcnt_hbm.at[pl.ds(s * CPT, CPT)],
                      cacc.at[pl.ds(s * CPT, CPT)])
      for j in range(C // 16):
        ones[pl.ds(j * 16, 16)] = jnp.ones((16,), jnp.float32)
      ones_t[...] = jnp.ones((TAIL,), jnp.float32)
    plsc.subcore_barrier()

    idx_start(base, bufs[0])
    idx_wait(base, bufs[0])
    gather_start(bufs[0])
    idx_start(base + C, bufs[1])

    def pair(i, carry):
      for j in range(2):
        k = 2 * i + j
        cur, nxt = bufs[j], bufs[1 - j]
        idx_wait(base + (k + 1) * C, nxt)
        gather_start(nxt)
        gather_wait(cur)
        scatter(cur)
        idx_start(base + (k + 2) * C, cur)
      return carry

    lax.fori_loop(0, (FULL - 2) // 2, pair, 0)

    off_t = base + FULL * C
    idx_wait(base + (FULL - 1) * C, bufs[1])
    gather_start(bufs[1])
    gather_wait(bufs[0])
    scatter(bufs[0])
    pltpu.async_copy(src_hbm.at[pl.ds(off_t, TAIL)], sidx_t, isem0)
    pltpu.async_copy(dst_hbm.at[pl.ds(off_t, TAIL)], didx_t, isem0)

    pltpu.make_async_copy(src_hbm.at[pl.ds(off_t, TAIL)], sidx_t, isem0).wait()
    pltpu.make_async_copy(dst_hbm.at[pl.ds(off_t, TAIL)], didx_t, isem0).wait()
    pltpu.async_copy(x_hbm.at[sidx_t], rows_t, gsem0)
    gather_wait(bufs[1])
    scatter(bufs[1])

    pltpu.make_async_copy(x_hbm.at[sidx_t], rows_t, gsem0).wait()
    pltpu.sync_copy(rows_t, acc.at[didx_t], add=True)
    if with_cnt:
      pltpu.sync_copy(ones_t, cacc.at[didx_t], add=True)

    plsc.subcore_barrier()

    pltpu.sync_copy(acc.at[pl.ds(row0, RPT)], p_hbm.at[c, pl.ds(row0, RPT)])
    if with_cnt:
      pltpu.sync_copy(cacc.at[pl.ds(s * CPT, CPT)],
                      cnt_hbm.at[c, pl.ds(s * CPT, CPT)])

  return pl.kernel(body, out_type=out_type, mesh=_mesh,
                   scratch_types=scratch)


_sc_agg_cnt = _make_sc_agg(True)
_sc_agg = _make_sc_agg(False)


def _matmul(xin, wt):
  def body(x_ref, w_ref, o_ref):
    o_ref[...] = jnp.dot(x_ref[...], w_ref[...],
                         preferred_element_type=jnp.float32)

  return pl.pallas_call(
      body, out_shape=jax.ShapeDtypeStruct((N, D), jnp.float32),
  )(xin, wt)


def _dense(p, inv, xr, wlt, bl, g, b, relu):
  def body(p_ref, inv_ref, xr_ref, wl_ref, bl_ref, g_ref, b_ref, o_ref):
    agg = p_ref[0, :N] + p_ref[1, :N]
    mean = agg * inv_ref[...]
    t = (jnp.dot(mean, wl_ref[...], preferred_element_type=jnp.float32)
         + xr_ref[...] + bl_ref[...])
    mu = jnp.mean(t, axis=0, keepdims=True)
    var = jnp.mean((t - mu) ** 2, axis=0, keepdims=True)
    h = (t - mu) * lax.rsqrt(var + 1e-5) * g_ref[...] + b_ref[...]
    if relu:
      h = jnp.maximum(h, 0.0)
    o_ref[...] = h

  return pl.pallas_call(
      body, out_shape=jax.ShapeDtypeStruct((N, D), jnp.float32),
  )(p, inv, xr, wlt, bl, g, b)


def kernel(x, edge_index, Wl1, bl1, Wr1, g1, b1, Wl2, bl2, Wr2, g2, b2):
  src = edge_index[0]
  dst = edge_index[1]
  zrows = jnp.zeros((RPT, D), jnp.float32)
  zcnt = jnp.zeros((NPAD,), jnp.float32)

  xr = _matmul(x, Wr1.T)
  p1, cnt = _sc_agg_cnt(src, dst, x, zrows, zcnt)
  cnt_tot = cnt[0, :N] + cnt[1, :N]
  inv = (1.0 / jnp.maximum(cnt_tot, 1.0))[:, None]

  h = _dense(p1, inv, xr, Wl1.T, bl1[None, :], g1[None, :], b1[None, :],
             relu=True)
  hr = _matmul(h, Wr2.T)
  (p2,) = _sc_agg(src, dst, h, zrows)
  out = _dense(p2, inv, hr, Wl2.T, bl2[None, :], g2[None, :],
               b2[None, :], relu=False)
  return out


# speedup vs baseline: 1.2022x; 1.2022x over previous
import functools

import jax
import jax.numpy as jnp
from jax import lax
from jax.experimental import pallas as pl
from jax.experimental.pallas import tpu as pltpu
from jax.experimental.pallas import tpu_sc as plsc

N = 10000
E = 320000
D = 128

NC = 2
NS = 16
NW = NC * NS
EPW = E // NW
C = 80
FULL = EPW // C
NPAD = 10240
RPT = NPAD // NS
CPT = NPAD // NS

_mesh = plsc.VectorSubcoreMesh(core_axis_name="c", subcore_axis_name="s")


def _make_sc_agg(with_cnt: bool):
  out_type = [jax.ShapeDtypeStruct((NC, NPAD, D), jnp.float32)]
  if with_cnt:
    out_type.append(jax.ShapeDtypeStruct((NC, NPAD), jnp.float32))
  NB = 4
  scratch = []
  for _ in range(NB):
    scratch += [
        pltpu.VMEM((C,), jnp.int32),
        pltpu.VMEM((C,), jnp.int32),
        pltpu.VMEM((C, D), jnp.float32),
        pltpu.SemaphoreType.DMA,
        pltpu.SemaphoreType.DMA,
        pltpu.SemaphoreType.DMA,
    ]
  scratch += [
      pltpu.VMEM_SHARED((NPAD, D), jnp.float32),
  ]
  if with_cnt:
    scratch += [
        pltpu.VMEM((C,), jnp.float32),
        pltpu.VMEM_SHARED((NPAD,), jnp.float32),
    ]

  NTB = NB * 6

  def body(src_hbm, dst_hbm, x_hbm, zrows_hbm, *rest):
    if with_cnt:
      zcnt_hbm, p_hbm, cnt_hbm = rest[:3]
      rest = rest[3:]
      ones, cacc = rest[NTB + 1:]
    else:
      p_hbm = rest[0]
      rest = rest[1:]
    bufs = tuple(tuple(rest[i * 6:i * 6 + 6]) for i in range(NB))
    acc = rest[NTB]
    c = lax.axis_index("c")
    s = lax.axis_index("s")
    wid = c * NS + s
    base = wid * EPW

    def idx_start(k, b):
      off = base + k * C
      pltpu.async_copy(src_hbm.at[pl.ds(off, C)], b[0], b[3])
      pltpu.async_copy(dst_hbm.at[pl.ds(off, C)], b[1], b[3])

    def idx_wait(k, b):
      off = base + k * C
      pltpu.make_async_copy(src_hbm.at[pl.ds(off, C)], b[0], b[3]).wait()
      pltpu.make_async_copy(dst_hbm.at[pl.ds(off, C)], b[1], b[3]).wait()

    def gather_start(b):
      pltpu.async_copy(x_hbm.at[b[0]], b[2], b[4])

    def gather_wait(b):
      pltpu.make_async_copy(x_hbm.at[b[0]], b[2], b[4]).wait()

    def scatter_start(b):
      pltpu.async_copy(b[2], acc.at[b[1]], b[5], add=True)
      if with_cnt:
        pltpu.async_copy(ones, cacc.at[b[1]], b[5], add=True)

    def scatter_wait(b):
      pltpu.make_async_copy(b[2], acc.at[b[1]], b[5]).wait()
      if with_cnt:
        pltpu.make_async_copy(ones, cacc.at[b[1]], b[5]).wait()

    row0 = s * RPT
    pltpu.sync_copy(zrows_hbm, acc.at[pl.ds(row0, RPT)])
    if with_cnt:
      pltpu.sync_copy(zcnt_hbm.at[pl.ds(s * CPT, CPT)],
                      cacc.at[pl.ds(s * CPT, CPT)])
      for j in range(C // 16):
        ones[pl.ds(j * 16, 16)] = jnp.ones((16,), jnp.float32)
    plsc.subcore_barrier()

    def steady(k, j):
      idx_wait(k, bufs[j])
      gather_start(bufs[j])
      scatter_wait(bufs[(j - 2) % NB])
      idx_start(k + 2, bufs[(j - 2) % NB])
      gather_wait(bufs[(j - 1) % NB])
      scatter_start(bufs[(j - 1) % NB])

    idx_start(0, bufs[0])
    idx_start(1, bufs[1])
    idx_wait(0, bufs[0])
    gather_start(bufs[0])
    idx_start(2, bufs[2])
    idx_wait(1, bufs[1])
    gather_start(bufs[1])
    idx_start(3, bufs[3])
    gather_wait(bufs[0])
    scatter_start(bufs[0])

    G, R = divmod(FULL - 4, NB)

    def group(i, carry):
      for j4 in range(NB):
        steady(2 + NB * i + j4, (2 + j4) % NB)
      return carry

    lax.fori_loop(0, G, group, 0)
    for r in range(R):
      k = 2 + NB * G + r
      steady(k, k % NB)

    kf = FULL - 2
    idx_wait(kf, bufs[kf % NB])
    gather_start(bufs[kf % NB])
    scatter_wait(bufs[(kf - 2) % NB])
    gather_wait(bufs[(kf - 1) % NB])
    scatter_start(bufs[(kf - 1) % NB])

    kf = FULL - 1
    idx_wait(kf, bufs[kf % NB])
    gather_start(bufs[kf % NB])
    scatter_wait(bufs[(kf - 2) % NB])
    gather_wait(bufs[(kf - 1) % NB])
    scatter_start(bufs[(kf - 1) % NB])

    gather_wait(bufs[(FULL - 1) % NB])
    scatter_start(bufs[(FULL - 1) % NB])
    scatter_wait(bufs[(FULL - 2) % NB])
    scatter_wait(bufs[(FULL - 1) % NB])

    plsc.subcore_barrier()

    pltpu.sync_copy(acc.at[pl.ds(row0, RPT)], p_hbm.at[c, pl.ds(row0, RPT)])
    if with_cnt:
      pltpu.sync_copy(cacc.at[pl.ds(s * CPT, CPT)],
                      cnt_hbm.at[c, pl.ds(s * CPT, CPT)])

  return pl.kernel(body, out_type=out_type, mesh=_mesh,
                   scratch_types=scratch)


_sc_agg_cnt = _make_sc_agg(True)
_sc_agg = _make_sc_agg(False)


def _dense(p, inv, xin, wlt, wrt, bl, g, b, relu):
  def body(p_ref, inv_ref, x_ref, wl_ref, wr_ref, bl_ref, g_ref, b_ref, o_ref):
    agg = p_ref[0, :N] + p_ref[1, :N]
    mean = agg * inv_ref[...]
    t = (jnp.dot(mean, wl_ref[...], preferred_element_type=jnp.float32)
         + jnp.dot(x_ref[...], wr_ref[...], preferred_element_type=jnp.float32)
         + bl_ref[...])
    mu = jnp.mean(t, axis=0, keepdims=True)
    var = jnp.mean((t - mu) ** 2, axis=0, keepdims=True)
    h = (t - mu) * lax.rsqrt(var + 1e-5) * g_ref[...] + b_ref[...]
    if relu:
      h = jnp.maximum(h, 0.0)
    o_ref[...] = h

  return pl.pallas_call(
      body, out_shape=jax.ShapeDtypeStruct((N, D), jnp.float32),
  )(p, inv, xin, wlt, wrt, bl, g, b)


def kernel(x, edge_index, Wl1, bl1, Wr1, g1, b1, Wl2, bl2, Wr2, g2, b2):
  src = edge_index[0]
  dst = edge_index[1]
  zrows = jnp.zeros((RPT, D), jnp.float32)
  zcnt = jnp.zeros((NPAD,), jnp.float32)

  p1, cnt = _sc_agg_cnt(src, dst, x, zrows, zcnt)
  cnt_tot = cnt[0, :N] + cnt[1, :N]
  inv = (1.0 / jnp.maximum(cnt_tot, 1.0))[:, None]

  h = _dense(p1, inv, x, Wl1.T, Wr1.T, bl1[None, :], g1[None, :], b1[None, :],
             relu=True)
  (p2,) = _sc_agg(src, dst, h, zrows)
  out = _dense(p2, inv, h, Wl2.T, Wr2.T, bl2[None, :], g2[None, :],
               b2[None, :], relu=False)
  return out


# speedup vs baseline: 1.2352x; 1.0274x over previous
import functools

import jax
import jax.numpy as jnp
from jax import lax
from jax.experimental import pallas as pl
from jax.experimental.pallas import tpu as pltpu
from jax.experimental.pallas import tpu_sc as plsc

N = 10000
E = 320000
D = 128

NC = 2
NS = 16
NW = NC * NS
EPW = E // NW
C = 80
FULL = EPW // C
NPAD = 10240
RPT = NPAD // NS
CPT = NPAD // NS

_mesh = plsc.VectorSubcoreMesh(core_axis_name="c", subcore_axis_name="s")


def _make_sc_agg(with_cnt: bool):
  out_type = [jax.ShapeDtypeStruct((NC, NPAD, D), jnp.float32)]
  if with_cnt:
    out_type.append(jax.ShapeDtypeStruct((NC, NPAD), jnp.float32))
  NB = 4
  scratch = []
  for _ in range(NB):
    scratch += [
        pltpu.VMEM((C,), jnp.int32),
        pltpu.VMEM((C,), jnp.int32),
        pltpu.VMEM((C, D), jnp.float32),
        pltpu.SemaphoreType.DMA,
        pltpu.SemaphoreType.DMA,
        pltpu.SemaphoreType.DMA,
    ]
  scratch += [
      pltpu.VMEM_SHARED((NPAD, D), jnp.float32),
  ]
  if with_cnt:
    scratch += [
        pltpu.VMEM((C,), jnp.float32),
        pltpu.VMEM_SHARED((NPAD,), jnp.float32),
    ]

  NTB = NB * 6

  def body(src_hbm, dst_hbm, x_hbm, zrows_hbm, *rest):
    if with_cnt:
      zcnt_hbm, p_hbm, cnt_hbm = rest[:3]
      rest = rest[3:]
      ones, cacc = rest[NTB + 1:]
    else:
      p_hbm = rest[0]
      rest = rest[1:]
    bufs = tuple(tuple(rest[i * 6:i * 6 + 6]) for i in range(NB))
    acc = rest[NTB]
    c = lax.axis_index("c")
    s = lax.axis_index("s")
    wid = c * NS + s
    base = wid * EPW

    def idx_start(k, b):
      off = base + k * C
      pltpu.async_copy(src_hbm.at[pl.ds(off, C)], b[0], b[3])
      pltpu.async_copy(dst_hbm.at[pl.ds(off, C)], b[1], b[3])

    def idx_wait(k, b):
      off = base + k * C
      pltpu.make_async_copy(src_hbm.at[pl.ds(off, C)], b[0], b[3]).wait()
      pltpu.make_async_copy(dst_hbm.at[pl.ds(off, C)], b[1], b[3]).wait()

    def gather_start(b):
      pltpu.async_copy(x_hbm.at[b[0]], b[2], b[4])

    def gather_wait(b):
      pltpu.make_async_copy(x_hbm.at[b[0]], b[2], b[4]).wait()

    def scatter_start(b):
      pltpu.async_copy(b[2], acc.at[b[1]], b[5], add=True)
      if with_cnt:
        pltpu.async_copy(ones, cacc.at[b[1]], b[5], add=True)

    def scatter_wait(b):
      pltpu.make_async_copy(b[2], acc.at[b[1]], b[5]).wait()
      if with_cnt:
        pltpu.make_async_copy(ones, cacc.at[b[1]], b[5]).wait()

    row0 = s * RPT
    pltpu.sync_copy(zrows_hbm, acc.at[pl.ds(row0, RPT)])
    if with_cnt:
      pltpu.sync_copy(zcnt_hbm.at[pl.ds(s * CPT, CPT)],
                      cacc.at[pl.ds(s * CPT, CPT)])
      for j in range(C // 16):
        ones[pl.ds(j * 16, 16)] = jnp.ones((16,), jnp.float32)
    plsc.subcore_barrier()

    def steady(k, j):
      idx_wait(k, bufs[j])
      gather_start(bufs[j])
      scatter_wait(bufs[(j - 2) % NB])
      idx_start(k + 2, bufs[(j - 2) % NB])
      gather_wait(bufs[(j - 1) % NB])
      scatter_start(bufs[(j - 1) % NB])

    idx_start(0, bufs[0])
    idx_start(1, bufs[1])
    idx_wait(0, bufs[0])
    gather_start(bufs[0])
    idx_start(2, bufs[2])
    idx_wait(1, bufs[1])
    gather_start(bufs[1])
    idx_start(3, bufs[3])
    gather_wait(bufs[0])
    scatter_start(bufs[0])

    G, R = divmod(FULL - 4, NB)

    def group(i, carry):
      for j4 in range(NB):
        steady(2 + NB * i + j4, (2 + j4) % NB)
      return carry

    lax.fori_loop(0, G, group, 0)
    for r in range(R):
      k = 2 + NB * G + r
      steady(k, k % NB)

    kf = FULL - 2
    idx_wait(kf, bufs[kf % NB])
    gather_start(bufs[kf % NB])
    scatter_wait(bufs[(kf - 2) % NB])
    gather_wait(bufs[(kf - 1) % NB])
    scatter_start(bufs[(kf - 1) % NB])

    kf = FULL - 1
    idx_wait(kf, bufs[kf % NB])
    gather_start(bufs[kf % NB])
    scatter_wait(bufs[(kf - 2) % NB])
    gather_wait(bufs[(kf - 1) % NB])
    scatter_start(bufs[(kf - 1) % NB])

    gather_wait(bufs[(FULL - 1) % NB])
    scatter_start(bufs[(FULL - 1) % NB])
    scatter_wait(bufs[(FULL - 2) % NB])
    scatter_wait(bufs[(FULL - 1) % NB])

    plsc.subcore_barrier()

    pltpu.sync_copy(acc.at[pl.ds(row0, RPT)], p_hbm.at[c, pl.ds(row0, RPT)])
    if with_cnt:
      pltpu.sync_copy(cacc.at[pl.ds(s * CPT, CPT)],
                      cnt_hbm.at[c, pl.ds(s * CPT, CPT)])

  return pl.kernel(body, out_type=out_type, mesh=_mesh,
                   scratch_types=scratch)


_sc_agg_cnt = _make_sc_agg(True)
_sc_agg = _make_sc_agg(False)


def _dense(p, cnt, xin, wlt, wrt, bl, g, b, relu):
  def body(p_ref, cnt_ref, x_ref, wl_ref, wr_ref, bl_ref, g_ref, b_ref, o_ref):
    agg = p_ref[0, :N] + p_ref[1, :N]
    cv = cnt_ref[0] + cnt_ref[1]
    inv = 1.0 / jnp.maximum(jnp.reshape(cv, (NPAD, 1))[:N], 1.0)
    mean = agg * inv
    t = (jnp.dot(mean, wl_ref[...], preferred_element_type=jnp.float32)
         + jnp.dot(x_ref[...], wr_ref[...], preferred_element_type=jnp.float32)
         + bl_ref[...])
    mu = jnp.mean(t, axis=0, keepdims=True)
    var = jnp.mean((t - mu) ** 2, axis=0, keepdims=True)
    h = (t - mu) * lax.rsqrt(var + 1e-5) * g_ref[...] + b_ref[...]
    if relu:
      h = jnp.maximum(h, 0.0)
    o_ref[...] = h

  return pl.pallas_call(
      body, out_shape=jax.ShapeDtypeStruct((N, D), jnp.float32),
  )(p, cnt, xin, wlt, wrt, bl, g, b)


def kernel(x, edge_index, Wl1, bl1, Wr1, g1, b1, Wl2, bl2, Wr2, g2, b2):
  src = edge_index[0]
  dst = edge_index[1]
  zrows = jnp.zeros((RPT, D), jnp.float32)
  zcnt = jnp.zeros((NPAD,), jnp.float32)

  p1, cnt = _sc_agg_cnt(src, dst, x, zrows, zcnt)

  h = _dense(p1, cnt, x, Wl1.T, Wr1.T, bl1[None, :], g1[None, :], b1[None, :],
             relu=True)
  (p2,) = _sc_agg(src, dst, h, zrows)
  out = _dense(p2, cnt, h, Wl2.T, Wr2.T, bl2[None, :], g2[None, :],
               b2[None, :], relu=False)
  return out


# speedup vs baseline: 1.2396x; 1.0035x over previous
import functools

import jax
import jax.numpy as jnp
from jax import lax
from jax.experimental import pallas as pl
from jax.experimental.pallas import tpu as pltpu
from jax.experimental.pallas import tpu_sc as plsc

N = 10000
E = 320000
D = 128

NC = 2
NS = 16
NW = NC * NS
EPW = E // NW
C = 80
FULL = EPW // C
NPAD = 10240
RPT = NPAD // NS
CPT = NPAD // NS

_mesh = plsc.VectorSubcoreMesh(core_axis_name="c", subcore_axis_name="s")


def _make_sc_agg(with_cnt: bool):
  out_type = [jax.ShapeDtypeStruct((NC, NPAD, D), jnp.float32)]
  if with_cnt:
    out_type.append(jax.ShapeDtypeStruct((NC, NPAD), jnp.float32))
  NB = 4
  scratch = []
  for _ in range(NB):
    scratch += [
        pltpu.VMEM((C,), jnp.int32),
        pltpu.VMEM((C,), jnp.int32),
        pltpu.VMEM((C, D), jnp.float32),
        pltpu.SemaphoreType.DMA,
        pltpu.SemaphoreType.DMA,
        pltpu.SemaphoreType.DMA,
    ]
  scratch += [
      pltpu.VMEM_SHARED((NPAD, D), jnp.float32),
  ]
  if with_cnt:
    scratch += [
        pltpu.VMEM((C,), jnp.float32),
        pltpu.VMEM_SHARED((NPAD,), jnp.float32),
    ]

  NTB = NB * 6

  def body(src_hbm, dst_hbm, x_hbm, zrows_hbm, *rest):
    if with_cnt:
      zcnt_hbm, p_hbm, cnt_hbm = rest[:3]
      rest = rest[3:]
      ones, cacc = rest[NTB + 1:]
    else:
      p_hbm = rest[0]
      rest = rest[1:]
    bufs = tuple(tuple(rest[i * 6:i * 6 + 6]) for i in range(NB))
    acc = rest[NTB]
    c = lax.axis_index("c")
    s = lax.axis_index("s")
    wid = c * NS + s
    base = wid * EPW

    def idx_start(k, b):
      off = base + k * C
      pltpu.async_copy(src_hbm.at[pl.ds(off, C)], b[0], b[3])
      pltpu.async_copy(dst_hbm.at[pl.ds(off, C)], b[1], b[3])

    def idx_wait(k, b):
      off = base + k * C
      pltpu.make_async_copy(src_hbm.at[pl.ds(off, C)], b[0], b[3]).wait()
      pltpu.make_async_copy(dst_hbm.at[pl.ds(off, C)], b[1], b[3]).wait()

    def gather_start(b):
      pltpu.async_copy(x_hbm.at[b[0]], b[2], b[4])

    def gather_wait(b):
      pltpu.make_async_copy(x_hbm.at[b[0]], b[2], b[4]).wait()

    def scatter_start(b):
      pltpu.async_copy(b[2], acc.at[b[1]], b[5], add=True)
      if with_cnt:
        pltpu.async_copy(ones, cacc.at[b[1]], b[5], add=True)

    def scatter_wait(b):
      pltpu.make_async_copy(b[2], acc.at[b[1]], b[5]).wait()
      if with_cnt:
        pltpu.make_async_copy(ones, cacc.at[b[1]], b[5]).wait()

    row0 = s * RPT
    pltpu.sync_copy(zrows_hbm, acc.at[pl.ds(row0, RPT)])
    if with_cnt:
      pltpu.sync_copy(zcnt_hbm.at[pl.ds(s * CPT, CPT)],
                      cacc.at[pl.ds(s * CPT, CPT)])
      for j in range(C // 16):
        ones[pl.ds(j * 16, 16)] = jnp.ones((16,), jnp.float32)
    plsc.subcore_barrier()

    def steady(k, j):
      idx_wait(k, bufs[j])
      gather_start(bufs[j])
      scatter_wait(bufs[(j - 2) % NB])
      idx_start(k + 2, bufs[(j - 2) % NB])
      gather_wait(bufs[(j - 1) % NB])
      scatter_start(bufs[(j - 1) % NB])

    idx_start(0, bufs[0])
    idx_start(1, bufs[1])
    idx_wait(0, bufs[0])
    gather_start(bufs[0])
    idx_start(2, bufs[2])
    idx_wait(1, bufs[1])
    gather_start(bufs[1])
    idx_start(3, bufs[3])
    gather_wait(bufs[0])
    scatter_start(bufs[0])

    G, R = divmod(FULL - 4, NB)

    def group(i, carry):
      for j4 in range(NB):
        steady(2 + NB * i + j4, (2 + j4) % NB)
      return carry

    lax.fori_loop(0, G, group, 0)
    for r in range(R):
      k = 2 + NB * G + r
      steady(k, k % NB)

    kf = FULL - 2
    idx_wait(kf, bufs[kf % NB])
    gather_start(bufs[kf % NB])
    scatter_wait(bufs[(kf - 2) % NB])
    gather_wait(bufs[(kf - 1) % NB])
    scatter_start(bufs[(kf - 1) % NB])

    kf = FULL - 1
    idx_wait(kf, bufs[kf % NB])
    gather_start(bufs[kf % NB])
    scatter_wait(bufs[(kf - 2) % NB])
    gather_wait(bufs[(kf - 1) % NB])
    scatter_start(bufs[(kf - 1) % NB])

    gather_wait(bufs[(FULL - 1) % NB])
    scatter_start(bufs[(FULL - 1) % NB])
    scatter_wait(bufs[(FULL - 2) % NB])
    scatter_wait(bufs[(FULL - 1) % NB])

    plsc.subcore_barrier()

    pltpu.sync_copy(acc.at[pl.ds(row0, RPT)], p_hbm.at[c, pl.ds(row0, RPT)])
    if with_cnt:
      pltpu.sync_copy(cacc.at[pl.ds(s * CPT, CPT)],
                      cnt_hbm.at[c, pl.ds(s * CPT, CPT)])

  return pl.kernel(body, out_type=out_type, mesh=_mesh,
                   scratch_types=scratch)


_sc_agg_cnt = _make_sc_agg(True)
_sc_agg = _make_sc_agg(False)


_DNT = (((1,), (1,)), ((), ()))


def _dense(p, cnt, xin, wl, wr, bl, g, b, relu):
  def body(p_ref, cnt_ref, x_ref, wl_ref, wr_ref, bl_ref, g_ref, b_ref, o_ref):
    agg = p_ref[0, :N] + p_ref[1, :N]
    cv = cnt_ref[0] + cnt_ref[1]
    inv = 1.0 / jnp.maximum(jnp.reshape(cv, (NPAD, 1))[:N], 1.0)
    mean = agg * inv
    t = (lax.dot_general(mean, wl_ref[...], _DNT,
                         preferred_element_type=jnp.float32)
         + lax.dot_general(x_ref[...], wr_ref[...], _DNT,
                           preferred_element_type=jnp.float32)
         + bl_ref[...])
    mu = jnp.mean(t, axis=0, keepdims=True)
    var = jnp.mean((t - mu) ** 2, axis=0, keepdims=True)
    h = (t - mu) * lax.rsqrt(var + 1e-5) * g_ref[...] + b_ref[...]
    if relu:
      h = jnp.maximum(h, 0.0)
    o_ref[...] = h

  return pl.pallas_call(
      body, out_shape=jax.ShapeDtypeStruct((N, D), jnp.float32),
  )(p, cnt, xin, wl, wr, bl, g, b)


def kernel(x, edge_index, Wl1, bl1, Wr1, g1, b1, Wl2, bl2, Wr2, g2, b2):
  src = edge_index[0]
  dst = edge_index[1]
  zrows = jnp.zeros((RPT, D), jnp.float32)
  zcnt = jnp.zeros((NPAD,), jnp.float32)

  p1, cnt = _sc_agg_cnt(src, dst, x, zrows, zcnt)

  h = _dense(p1, cnt, x, Wl1, Wr1, bl1, g1, b1, relu=True)
  (p2,) = _sc_agg(src, dst, h, zrows)
  out = _dense(p2, cnt, h, Wl2, Wr2, bl2, g2, b2, relu=False)
  return out
